# Initial kernel scaffold; baseline (speedup 1.0000x reference)
import jax
import jax.numpy as jnp
from jax.experimental import pallas as pl


def kernel(input, prompt, W, b):
    raise NotImplementedError("write your pallas kernel here")


# speedup vs baseline: 1.1545x; 1.1545x over previous
import jax
import jax.numpy as jnp
from jax.experimental import pallas as pl
from jax.experimental.pallas import tpu as pltpu

MODEL_DIM = 4096
PROMPT_DIM = 64
NUM_EXPERTS = 64
T_TILE = 256


def _gate_kernel(x_ref, p_ref, wm_ref, wp_ref, b_ref,
                 m0_ref, m1_ref, g0_ref, g1_ref):
    x = x_ref[...]
    p = p_ref[...]
    logits = jax.lax.dot_general(
        x, wm_ref[...], (((1,), (1,)), ((), ())),
        precision=jax.lax.Precision.DEFAULT)
    logits = logits + jax.lax.dot_general(
        p, wp_ref[...], (((1,), (1,)), ((), ())),
        precision=jax.lax.Precision.DEFAULT)
    logits = logits + b_ref[...]

    rows = logits.shape[0]
    cols = jax.lax.broadcasted_iota(jnp.int32, (rows, NUM_EXPERTS), 1)
    i1 = jnp.argmax(logits, axis=1)
    onehot1 = cols == i1[:, None]
    l1 = jnp.max(logits, axis=1, keepdims=True)
    masked = jnp.where(onehot1, -jnp.inf, logits)
    i2 = jnp.argmax(masked, axis=1)
    onehot2 = cols == i2[:, None]
    l2 = jnp.max(masked, axis=1, keepdims=True)

    e = jnp.exp(l2 - l1)
    g0 = 1.0 / (1.0 + e)
    m0_ref[...] = onehot1.astype(jnp.int32)
    m1_ref[...] = onehot2.astype(jnp.int32)
    g0_ref[...] = g0
    g1_ref[...] = 1.0 - g0


def kernel(input, prompt, W, b):
    T = input.shape[0]
    Wm = W[:, :MODEL_DIM]
    Wp = W[:, MODEL_DIM:]
    b2 = b.reshape(1, NUM_EXPERTS)
    grid = (T // T_TILE,)
    m0, m1, g0, g1 = pl.pallas_call(
        _gate_kernel,
        grid=grid,
        in_specs=[
            pl.BlockSpec((T_TILE, MODEL_DIM), lambda i: (i, 0)),
            pl.BlockSpec((T_TILE, PROMPT_DIM), lambda i: (i, 0)),
            pl.BlockSpec((NUM_EXPERTS, MODEL_DIM), lambda i: (0, 0)),
            pl.BlockSpec((NUM_EXPERTS, PROMPT_DIM), lambda i: (0, 0)),
            pl.BlockSpec((1, NUM_EXPERTS), lambda i: (0, 0)),
        ],
        out_specs=[
            pl.BlockSpec((T_TILE, NUM_EXPERTS), lambda i: (i, 0)),
            pl.BlockSpec((T_TILE, NUM_EXPERTS), lambda i: (i, 0)),
            pl.BlockSpec((T_TILE, 1), lambda i: (i, 0)),
            pl.BlockSpec((T_TILE, 1), lambda i: (i, 0)),
        ],
        out_shape=[
            jax.ShapeDtypeStruct((T, NUM_EXPERTS), jnp.int32),
            jax.ShapeDtypeStruct((T, NUM_EXPERTS), jnp.int32),
            jax.ShapeDtypeStruct((T, 1), jnp.float32),
            jax.ShapeDtypeStruct((T, 1), jnp.float32),
        ],
        compiler_params=pltpu.CompilerParams(
            dimension_semantics=("arbitrary",)),
    )(input, prompt, Wm, Wp, b2)
    return m0, m1, g0.reshape(T), g1.reshape(T)


# speedup vs baseline: 1.3370x; 1.1581x over previous
import jax
import jax.numpy as jnp
from jax.experimental import pallas as pl
from jax.experimental.pallas import tpu as pltpu

MODEL_DIM = 4096
PROMPT_DIM = 64
NUM_EXPERTS = 64
T_TILE = 512


def _gate_kernel(x_ref, p_ref, wm_ref, wp_ref, b_ref,
                 m0_ref, m1_ref, g0_ref, g1_ref):
    x = x_ref[...]
    p = p_ref[...]
    logits = jax.lax.dot_general(
        x, wm_ref[...], (((1,), (1,)), ((), ())),
        precision=jax.lax.Precision.DEFAULT)
    logits = logits + jax.lax.dot_general(
        p, wp_ref[...], (((1,), (1,)), ((), ())),
        precision=jax.lax.Precision.DEFAULT)
    logits = logits + b_ref[...]

    rows = logits.shape[0]
    cols = jax.lax.broadcasted_iota(jnp.int32, (rows, NUM_EXPERTS), 1)
    i1 = jnp.argmax(logits, axis=1)
    onehot1 = cols == i1[:, None]
    l1 = jnp.max(logits, axis=1, keepdims=True)
    masked = jnp.where(onehot1, -jnp.inf, logits)
    i2 = jnp.argmax(masked, axis=1)
    onehot2 = cols == i2[:, None]
    l2 = jnp.max(masked, axis=1, keepdims=True)

    e = jnp.exp(l2 - l1)
    g0 = 1.0 / (1.0 + e)
    m0_ref[...] = onehot1.astype(jnp.int32)
    m1_ref[...] = onehot2.astype(jnp.int32)
    g0_ref[...] = g0
    g1_ref[...] = 1.0 - g0


def kernel(input, prompt, W, b):
    T = input.shape[0]
    Wm = W[:, :MODEL_DIM]
    Wp = W[:, MODEL_DIM:]
    b2 = b.reshape(1, NUM_EXPERTS)
    grid = (T // T_TILE,)
    m0, m1, g0, g1 = pl.pallas_call(
        _gate_kernel,
        grid=grid,
        in_specs=[
            pl.BlockSpec((T_TILE, MODEL_DIM), lambda i: (i, 0)),
            pl.BlockSpec((T_TILE, PROMPT_DIM), lambda i: (i, 0)),
            pl.BlockSpec((NUM_EXPERTS, MODEL_DIM), lambda i: (0, 0)),
            pl.BlockSpec((NUM_EXPERTS, PROMPT_DIM), lambda i: (0, 0)),
            pl.BlockSpec((1, NUM_EXPERTS), lambda i: (0, 0)),
        ],
        out_specs=[
            pl.BlockSpec((T_TILE, NUM_EXPERTS), lambda i: (i, 0)),
            pl.BlockSpec((T_TILE, NUM_EXPERTS), lambda i: (i, 0)),
            pl.BlockSpec((T_TILE, 1), lambda i: (i, 0)),
            pl.BlockSpec((T_TILE, 1), lambda i: (i, 0)),
        ],
        out_shape=[
            jax.ShapeDtypeStruct((T, NUM_EXPERTS), jnp.int32),
            jax.ShapeDtypeStruct((T, NUM_EXPERTS), jnp.int32),
            jax.ShapeDtypeStruct((T, 1), jnp.float32),
            jax.ShapeDtypeStruct((T, 1), jnp.float32),
        ],
        compiler_params=pltpu.CompilerParams(
            dimension_semantics=("arbitrary",)),
    )(input, prompt, Wm, Wp, b2)
    return m0, m1, g0.reshape(T), g1.reshape(T)


# speedup vs baseline: 1.3538x; 1.0126x over previous
import jax
import jax.numpy as jnp
from jax.experimental import pallas as pl
from jax.experimental.pallas import tpu as pltpu

MODEL_DIM = 4096
PROMPT_DIM = 64
NUM_EXPERTS = 64
T_TILE = 1024


def _gate_kernel(x_ref, p_ref, wm_ref, wp_ref, b_ref,
                 m0_ref, m1_ref, g0_ref, g1_ref):
    x = x_ref[...]
    p = p_ref[...]
    logits = jax.lax.dot_general(
        x, wm_ref[...], (((1,), (1,)), ((), ())),
        precision=jax.lax.Precision.DEFAULT)
    logits = logits + jax.lax.dot_general(
        p, wp_ref[...], (((1,), (1,)), ((), ())),
        precision=jax.lax.Precision.DEFAULT)
    logits = logits + b_ref[...]

    rows = logits.shape[0]
    cols = jax.lax.broadcasted_iota(jnp.int32, (rows, NUM_EXPERTS), 1)
    i1 = jnp.argmax(logits, axis=1)
    onehot1 = cols == i1[:, None]
    l1 = jnp.max(logits, axis=1, keepdims=True)
    masked = jnp.where(onehot1, -jnp.inf, logits)
    i2 = jnp.argmax(masked, axis=1)
    onehot2 = cols == i2[:, None]
    l2 = jnp.max(masked, axis=1, keepdims=True)

    e = jnp.exp(l2 - l1)
    g0 = 1.0 / (1.0 + e)
    m0_ref[...] = onehot1.astype(jnp.int32)
    m1_ref[...] = onehot2.astype(jnp.int32)
    g0_ref[...] = g0
    g1_ref[...] = 1.0 - g0


def kernel(input, prompt, W, b):
    T = input.shape[0]
    Wm = W[:, :MODEL_DIM]
    Wp = W[:, MODEL_DIM:]
    b2 = b.reshape(1, NUM_EXPERTS)
    grid = (T // T_TILE,)
    m0, m1, g0, g1 = pl.pallas_call(
        _gate_kernel,
        grid=grid,
        in_specs=[
            pl.BlockSpec((T_TILE, MODEL_DIM), lambda i: (i, 0)),
            pl.BlockSpec((T_TILE, PROMPT_DIM), lambda i: (i, 0)),
            pl.BlockSpec((NUM_EXPERTS, MODEL_DIM), lambda i: (0, 0)),
            pl.BlockSpec((NUM_EXPERTS, PROMPT_DIM), lambda i: (0, 0)),
            pl.BlockSpec((1, NUM_EXPERTS), lambda i: (0, 0)),
        ],
        out_specs=[
            pl.BlockSpec((T_TILE, NUM_EXPERTS), lambda i: (i, 0)),
            pl.BlockSpec((T_TILE, NUM_EXPERTS), lambda i: (i, 0)),
            pl.BlockSpec((T_TILE, 1), lambda i: (i, 0)),
            pl.BlockSpec((T_TILE, 1), lambda i: (i, 0)),
        ],
        out_shape=[
            jax.ShapeDtypeStruct((T, NUM_EXPERTS), jnp.int32),
            jax.ShapeDtypeStruct((T, NUM_EXPERTS), jnp.int32),
            jax.ShapeDtypeStruct((T, 1), jnp.float32),
            jax.ShapeDtypeStruct((T, 1), jnp.float32),
        ],
        compiler_params=pltpu.CompilerParams(
            dimension_semantics=("arbitrary",)),
    )(input, prompt, Wm, Wp, b2)
    return m0, m1, g0.reshape(T), g1.reshape(T)
